# baseline (speedup 1.0000x reference)
import functools

import jax
import jax.numpy as jnp
from jax import lax
from jax.experimental import pallas as pl
from jax.experimental.pallas import tpu as pltpu
from jax.experimental.pallas import tpu_sc as plsc

NC = 2
NS = 16
NW = NC * NS
C = 64
RB = 25


def _sc_body(N, E, u_hbm, src_hbm, dst_hbm, w_hbm, out_hbm,
             idx_s0, idx_t0, idx_s1, idx_t1, us0, ut0, us1, ut1,
             wv0, wv1, ws, wt,
             rowbuf, ov, b_sh, sem_g0, sem_g1, sem_w0, sem_w1):
    nchunk_tot = E // C
    main = nchunk_tot // NW
    extra = nchunk_tot % NW
    npairs = main // 2
    rpt = N // NS
    nbatch = rpt // RB
    cid = lax.axis_index("c")
    sid = lax.axis_index("s")
    wid = sid * NC + cid
    iota16 = lax.iota(jnp.int32, 16)
    zf = jnp.zeros((16,), jnp.float32)

    idx = ((idx_s0, idx_t0), (idx_s1, idx_t1))
    us = (us0, us1)
    ut = (ut0, ut1)
    wv = (wv0, wv1)
    sem_g = (sem_g0, sem_g1)
    sem_w = (sem_w0, sem_w1)

    def issue(b, i):
        base = jnp.where(i < main, (wid * main + i) * C,
                         jnp.minimum((main * NW + wid) * C, E - C))
        pltpu.sync_copy(src_hbm.at[pl.ds(base, C)], idx[b][0])
        pltpu.sync_copy(dst_hbm.at[pl.ds(base, C)], idx[b][1])
        pltpu.async_copy(u_hbm.at[idx[b][0]], us[b], sem_g[b])
        pltpu.async_copy(u_hbm.at[idx[b][1]], ut[b], sem_g[b])
        pltpu.async_copy(w_hbm.at[pl.ds(base, C)], wv[b], sem_w[b])

    def wait_gathers(b):
        pltpu.make_async_copy(u_hbm.at[idx[b][0]], us[b], sem_g[b]).wait()
        pltpu.make_async_copy(u_hbm.at[idx[b][1]], ut[b], sem_g[b]).wait()

    def wait_w(b):
        pltpu.make_async_copy(w_hbm.at[pl.ds(0, C)], wv[b], sem_w[b]).wait()

    def compute(b, flag=None):
        wait_gathers(b)
        wait_w(b)

        rot = (iota16 * 17) & 127
        rot64 = (iota16 * 17) & 63

        def group_body(gi, gcarry):
            rows = gi * 16 + iota16

            def sixteen(o, accs):
                a0, a1, b0, b1, c0, c1 = accs
                cb = jnp.full((16,), o * 16, jnp.int32) + rot
                for jj in range(16):
                    cx = (cb + jj) & 127
                    cv = cx + 128
                    xs = plsc.load_gather(us[b], [rows, cx])
                    xt = plsc.load_gather(ut[b], [rows, cx])
                    vs = plsc.load_gather(us[b], [rows, cv])
                    vt = plsc.load_gather(ut[b], [rows, cv])
                    dd = xt - xs
                    if jj % 2 == 0:
                        a0 = a0 + dd * dd
                        b0 = b0 + vs * dd
                        c0 = c0 + vt * dd
                    else:
                        a1 = a1 + dd * dd
                        b1 = b1 + vs * dd
                        c1 = c1 + vt * dd
                return (a0, a1, b0, b1, c0, c1)

            a0, a1, b0, b1, c0, c1 = lax.fori_loop(
                0, 8, sixteen, (zf, zf, zf, zf, zf, zf))
            inv = 1.0 / jnp.maximum(a0 + a1, 1e-6)
            if flag is not None:
                inv = inv * flag
            csv = (b0 + b1) * inv
            ctv = (c0 + c1) * inv

            def wquad(m, wcarry):
                cb = jnp.full((16,), m * 4, jnp.int32) + rot64
                for jj in range(4):
                    cc = (cb + jj) & 63
                    wcol = plsc.load_gather(wv[b], [rows, cc])
                    plsc.store_scatter(ws, [rows, cc], wcol * csv)
                    plsc.store_scatter(wt, [rows, cc], wcol * ctv)
                return wcarry

            lax.fori_loop(0, 16, wquad, 0)
            return gcarry

        lax.fori_loop(0, C // 16, group_body, 0)
        pltpu.sync_copy(ws, b_sh.at[idx[b][0]], add=True)
        pltpu.sync_copy(wt, b_sh.at[idx[b][1]], add=True)

    issue(0, 0)
    zero16 = jnp.zeros((16,), jnp.float32)

    def zrow(r, carry):
        for g in range(4):
            rowbuf[r, pl.ds(g * 16, 16)] = zero16
        return carry

    lax.fori_loop(0, RB, zrow, 0)
    row0 = sid * rpt

    def zbatch(j, carry):
        pltpu.sync_copy(rowbuf, b_sh.at[pl.ds(row0 + j * RB, RB)])
        return carry

    lax.fori_loop(0, nbatch, zbatch, 0)
    plsc.subcore_barrier()

    issue(1, 1)

    def pair(j, carry):
        compute(0)
        issue(0, 2 * j + 2)
        compute(1)
        issue(1, 2 * j + 3)
        return carry

    lax.fori_loop(0, npairs, pair, 0)
    compute(0, flag=jnp.where(wid < extra, 1.0, 0.0).astype(jnp.float32))
    wait_gathers(1)
    wait_w(1)
    plsc.subcore_barrier()

    perms = []
    for g in range(4):
        fl = iota16 + (g * 16)
        i8 = fl // 8
        j8 = fl % 8
        perms.append(j8 * 8 + i8)

    def abatch(j, carry):
        pltpu.sync_copy(b_sh.at[pl.ds(row0 + j * RB, RB)], rowbuf)

        def arow(r, rcarry):
            rsplat = jnp.full((16,), r, jnp.int32)
            for g in range(4):
                bvec = rowbuf[r, pl.ds(g * 16, 16)]
                btv = plsc.load_gather(rowbuf, [rsplat, perms[g]])
                ov[pl.ds(r * 64 + g * 16, 16)] = 0.5 * (bvec - btv)
            return rcarry

        lax.fori_loop(0, RB, arow, 0)
        pltpu.sync_copy(
            ov,
            out_hbm.at[pl.ds((cid * N + row0 + j * RB) * 64, RB * 64)])
        return carry

    lax.fori_loop(0, nbatch, abatch, 0)


@functools.lru_cache(maxsize=None)
def _make_sc(N, E):
    body = functools.partial(_sc_body, N, E)
    return pl.kernel(
        body,
        out_type=jax.ShapeDtypeStruct((NC * N * 64,), jnp.float32),
        mesh=plsc.VectorSubcoreMesh(
            core_axis_name="c", subcore_axis_name="s",
            num_cores=NC, num_subcores=NS),
        scratch_types=[
            pltpu.VMEM((C,), jnp.int32),
            pltpu.VMEM((C,), jnp.int32),
            pltpu.VMEM((C,), jnp.int32),
            pltpu.VMEM((C,), jnp.int32),
            pltpu.VMEM((C, 256), jnp.float32),
            pltpu.VMEM((C, 256), jnp.float32),
            pltpu.VMEM((C, 256), jnp.float32),
            pltpu.VMEM((C, 256), jnp.float32),
            pltpu.VMEM((C, 64), jnp.float32),
            pltpu.VMEM((C, 64), jnp.float32),
            pltpu.VMEM((C, 64), jnp.float32),
            pltpu.VMEM((C, 64), jnp.float32),
            pltpu.VMEM((RB, 64), jnp.float32),
            pltpu.VMEM((RB * 64,), jnp.float32),
            pltpu.VMEM_SHARED((N, 64), jnp.float32),
            pltpu.SemaphoreType.DMA,
            pltpu.SemaphoreType.DMA,
            pltpu.SemaphoreType.DMA,
            pltpu.SemaphoreType.DMA,
        ],
        compiler_params=pltpu.CompilerParams(
            use_tc_tiling_on_sc=False, needs_layout_passes=False),
    )


def _combine_body(p_ref, o_ref):
    o_ref[...] = p_ref[0] + p_ref[1]


@functools.lru_cache(maxsize=None)
def _make_combine(R):
    br = R // 5
    return pl.pallas_call(
        _combine_body,
        out_shape=jax.ShapeDtypeStruct((R, 128), jnp.float32),
        grid=(R // br,),
        in_specs=[pl.BlockSpec((2, br, 128), lambda i: (0, i, 0))],
        out_specs=pl.BlockSpec((br, 128), lambda i: (i, 0)),
    )


@jax.jit
def kernel(x, v, edges, omega_params):
    N, D = x.shape
    E = edges.shape[0]
    K = omega_params.shape[1]
    assert D == 128 and K == 8
    assert E % C == 0 and N % (NS * RB) == 0
    assert ((E // C) // NW) % 2 == 0

    u = jnp.concatenate([x, v], axis=1)
    src = edges[:, 0]
    dst = edges[:, 1]
    w = omega_params.reshape(E, K * K)

    partial = _make_sc(N, E)(u, src, dst, w)
    r = (N * K * K) // 128
    summed = _make_combine(r)(partial.reshape(2, r, 128))
    return summed.reshape(N, K, K)


# speedup vs baseline: 1.0209x; 1.0209x over previous
import functools

import jax
import jax.numpy as jnp
from jax import lax
from jax.experimental import pallas as pl
from jax.experimental.pallas import tpu as pltpu
from jax.experimental.pallas import tpu_sc as plsc

NC = 2
NS = 16
NW = NC * NS
C = 64
RB = 25


def _sc_body(N, E, u_hbm, src_hbm, dst_hbm, w_hbm, out_hbm,
             idx_s0, idx_t0, idx_s1, idx_t1, us0, ut0, us1, ut1,
             wv0, wv1, ws, wt, pa, pb, pc,
             rowbuf, ov, b_sh, sem_g0, sem_g1, sem_w0, sem_w1):
    nchunk_tot = E // C
    main = nchunk_tot // NW
    extra = nchunk_tot % NW
    npairs = main // 2
    rpt = N // NS
    nbatch = rpt // RB
    cid = lax.axis_index("c")
    sid = lax.axis_index("s")
    wid = sid * NC + cid
    iota16 = lax.iota(jnp.int32, 16)
    zf = jnp.zeros((16,), jnp.float32)

    idx = ((idx_s0, idx_t0), (idx_s1, idx_t1))
    us = (us0, us1)
    ut = (ut0, ut1)
    wv = (wv0, wv1)
    sem_g = (sem_g0, sem_g1)
    sem_w = (sem_w0, sem_w1)

    def issue(b, i):
        base = jnp.where(i < main, (wid * main + i) * C,
                         jnp.minimum((main * NW + wid) * C, E - C))
        pltpu.sync_copy(src_hbm.at[pl.ds(base, C)], idx[b][0])
        pltpu.sync_copy(dst_hbm.at[pl.ds(base, C)], idx[b][1])
        pltpu.async_copy(u_hbm.at[idx[b][0]], us[b], sem_g[b])
        pltpu.async_copy(u_hbm.at[idx[b][1]], ut[b], sem_g[b])
        pltpu.async_copy(w_hbm.at[pl.ds(base, C)], wv[b], sem_w[b])

    def wait_gathers(b):
        pltpu.make_async_copy(u_hbm.at[idx[b][0]], us[b], sem_g[b]).wait()
        pltpu.make_async_copy(u_hbm.at[idx[b][1]], ut[b], sem_g[b]).wait()

    def wait_w(b):
        pltpu.make_async_copy(w_hbm.at[pl.ds(0, C)], wv[b], sem_w[b]).wait()

    def compute(b, flag=None):
        wait_gathers(b)
        wait_w(b)

        def group_body(gi, gcarry):
            e0 = gi * 16

            for jj in range(16):
                e = e0 + jj
                acc_a = zf
                acc_b = zf
                acc_c = zf
                for k in range(8):
                    xs = us[b][e, pl.ds(k * 16, 16)]
                    xt = ut[b][e, pl.ds(k * 16, 16)]
                    vs = us[b][e, pl.ds(128 + k * 16, 16)]
                    vt = ut[b][e, pl.ds(128 + k * 16, 16)]
                    dd = xt - xs
                    acc_a = acc_a + dd * dd
                    acc_b = acc_b + vs * dd
                    acc_c = acc_c + vt * dd
                pa[jj, pl.ds(0, 16)] = acc_a
                pb[jj, pl.ds(0, 16)] = acc_b
                pc[jj, pl.ds(0, 16)] = acc_c

            ta = zf
            tb = zf
            tc = zf
            for j in range(16):
                jsp = jnp.full((16,), j, jnp.int32)
                ta = ta + plsc.load_gather(pa, [iota16, jsp])
                tb = tb + plsc.load_gather(pb, [iota16, jsp])
                tc = tc + plsc.load_gather(pc, [iota16, jsp])
            inv = 1.0 / jnp.maximum(ta, 1e-6)
            if flag is not None:
                inv = inv * flag
            csv = tb * inv
            ctv = tc * inv

            for jj in range(16):
                e = e0 + jj
                css = csv[jj]
                cts = ctv[jj]
                for g in range(4):
                    wrow = wv[b][e, pl.ds(g * 16, 16)]
                    ws[e, pl.ds(g * 16, 16)] = wrow * css
                    wt[e, pl.ds(g * 16, 16)] = wrow * cts
            return gcarry

        lax.fori_loop(0, C // 16, group_body, 0)
        pltpu.sync_copy(ws, b_sh.at[idx[b][0]], add=True)
        pltpu.sync_copy(wt, b_sh.at[idx[b][1]], add=True)

    issue(0, 0)
    zero16 = jnp.zeros((16,), jnp.float32)

    def zrow(r, carry):
        for g in range(4):
            rowbuf[r, pl.ds(g * 16, 16)] = zero16
        return carry

    lax.fori_loop(0, RB, zrow, 0)
    row0 = sid * rpt

    def zbatch(j, carry):
        pltpu.sync_copy(rowbuf, b_sh.at[pl.ds(row0 + j * RB, RB)])
        return carry

    lax.fori_loop(0, nbatch, zbatch, 0)
    plsc.subcore_barrier()

    issue(1, 1)

    def pair(j, carry):
        compute(0)
        issue(0, 2 * j + 2)
        compute(1)
        issue(1, 2 * j + 3)
        return carry

    lax.fori_loop(0, npairs, pair, 0)
    compute(0, flag=jnp.where(wid < extra, 1.0, 0.0).astype(jnp.float32))
    wait_gathers(1)
    wait_w(1)
    plsc.subcore_barrier()

    perms = []
    for g in range(4):
        fl = iota16 + (g * 16)
        i8 = fl // 8
        j8 = fl % 8
        perms.append(j8 * 8 + i8)

    def abatch(j, carry):
        pltpu.sync_copy(b_sh.at[pl.ds(row0 + j * RB, RB)], rowbuf)

        def arow(r, rcarry):
            rsplat = jnp.full((16,), r, jnp.int32)
            for g in range(4):
                bvec = rowbuf[r, pl.ds(g * 16, 16)]
                btv = plsc.load_gather(rowbuf, [rsplat, perms[g]])
                ov[pl.ds(r * 64 + g * 16, 16)] = 0.5 * (bvec - btv)
            return rcarry

        lax.fori_loop(0, RB, arow, 0)
        pltpu.sync_copy(
            ov,
            out_hbm.at[pl.ds((cid * N + row0 + j * RB) * 64, RB * 64)])
        return carry

    lax.fori_loop(0, nbatch, abatch, 0)


@functools.lru_cache(maxsize=None)
def _make_sc(N, E):
    body = functools.partial(_sc_body, N, E)
    return pl.kernel(
        body,
        out_type=jax.ShapeDtypeStruct((NC * N * 64,), jnp.float32),
        mesh=plsc.VectorSubcoreMesh(
            core_axis_name="c", subcore_axis_name="s",
            num_cores=NC, num_subcores=NS),
        scratch_types=[
            pltpu.VMEM((C,), jnp.int32),
            pltpu.VMEM((C,), jnp.int32),
            pltpu.VMEM((C,), jnp.int32),
            pltpu.VMEM((C,), jnp.int32),
            pltpu.VMEM((C, 256), jnp.float32),
            pltpu.VMEM((C, 256), jnp.float32),
            pltpu.VMEM((C, 256), jnp.float32),
            pltpu.VMEM((C, 256), jnp.float32),
            pltpu.VMEM((C, 64), jnp.float32),
            pltpu.VMEM((C, 64), jnp.float32),
            pltpu.VMEM((C, 64), jnp.float32),
            pltpu.VMEM((C, 64), jnp.float32),
            pltpu.VMEM((16, 17), jnp.float32),
            pltpu.VMEM((16, 17), jnp.float32),
            pltpu.VMEM((16, 17), jnp.float32),
            pltpu.VMEM((RB, 64), jnp.float32),
            pltpu.VMEM((RB * 64,), jnp.float32),
            pltpu.VMEM_SHARED((N, 64), jnp.float32),
            pltpu.SemaphoreType.DMA,
            pltpu.SemaphoreType.DMA,
            pltpu.SemaphoreType.DMA,
            pltpu.SemaphoreType.DMA,
        ],
        compiler_params=pltpu.CompilerParams(
            use_tc_tiling_on_sc=False, needs_layout_passes=False),
    )


def _combine_body(p_ref, o_ref):
    o_ref[...] = p_ref[0] + p_ref[1]


@functools.lru_cache(maxsize=None)
def _make_combine(R):
    br = R // 5
    return pl.pallas_call(
        _combine_body,
        out_shape=jax.ShapeDtypeStruct((R, 128), jnp.float32),
        grid=(R // br,),
        in_specs=[pl.BlockSpec((2, br, 128), lambda i: (0, i, 0))],
        out_specs=pl.BlockSpec((br, 128), lambda i: (i, 0)),
    )


@jax.jit
def kernel(x, v, edges, omega_params):
    N, D = x.shape
    E = edges.shape[0]
    K = omega_params.shape[1]
    assert D == 128 and K == 8
    assert E % C == 0 and N % (NS * RB) == 0
    assert ((E // C) // NW) % 2 == 0

    u = jnp.concatenate([x, v], axis=1)
    src = edges[:, 0]
    dst = edges[:, 1]
    w = omega_params.reshape(E, K * K)

    partial = _make_sc(N, E)(u, src, dst, w)
    r = (N * K * K) // 128
    summed = _make_combine(r)(partial.reshape(2, r, 128))
    return summed.reshape(N, K, K)


# speedup vs baseline: 1.1728x; 1.1488x over previous
import functools

import jax
import jax.numpy as jnp
from jax import lax
from jax.experimental import pallas as pl
from jax.experimental.pallas import tpu as pltpu
from jax.experimental.pallas import tpu_sc as plsc

NC = 2
NS = 16
NW = NC * NS
C = 64
RB = 25


def _sc_body(N, E, u_hbm, src_hbm, dst_hbm, w_hbm, out_hbm,
             idx_s0, idx_t0, idx_s1, idx_t1,
             nxt_s0, nxt_t0, nxt_s1, nxt_t1, us0, ut0, us1, ut1,
             wv0, wv1, ws, wt, pa, pb, pc,
             rowbuf, ov, b_sh, sem_g0, sem_g1, sem_w0, sem_w1,
             sem_i0, sem_i1):
    nchunk_tot = E // C
    main = nchunk_tot // NW
    extra = nchunk_tot % NW
    npairs = main // 2
    rpt = N // NS
    nbatch = rpt // RB
    cid = lax.axis_index("c")
    sid = lax.axis_index("s")
    wid = sid * NC + cid
    iota16 = lax.iota(jnp.int32, 16)
    zf = jnp.zeros((16,), jnp.float32)

    idx = ((idx_s0, idx_t0), (idx_s1, idx_t1))
    nxt = ((nxt_s0, nxt_t0), (nxt_s1, nxt_t1))
    us = (us0, us1)
    ut = (ut0, ut1)
    wv = (wv0, wv1)
    sem_g = (sem_g0, sem_g1)
    sem_w = (sem_w0, sem_w1)
    sem_i = (sem_i0, sem_i1)

    def chunk_base(i):
        return jnp.where(i < main, (wid * main + i) * C,
                         jnp.minimum((main * NW + wid) * C, E - C))

    def fire(b, i):
        base = chunk_base(i)
        pltpu.async_copy(u_hbm.at[idx[b][0]], us[b], sem_g[b])
        pltpu.async_copy(u_hbm.at[idx[b][1]], ut[b], sem_g[b])
        pltpu.async_copy(w_hbm.at[pl.ds(base, C)], wv[b], sem_w[b])

    def prefetch_idx(b, i):
        base = chunk_base(i)
        pltpu.async_copy(src_hbm.at[pl.ds(base, C)], nxt[b][0], sem_i[b])
        pltpu.async_copy(dst_hbm.at[pl.ds(base, C)], nxt[b][1], sem_i[b])

    def wait_idx(b):
        pltpu.make_async_copy(
            src_hbm.at[pl.ds(0, C)], nxt[b][0], sem_i[b]).wait()
        pltpu.make_async_copy(
            dst_hbm.at[pl.ds(0, C)], nxt[b][1], sem_i[b]).wait()

    def advance(b, i):
        wait_idx(b)
        for q in range(C // 16):
            idx[b][0][pl.ds(q * 16, 16)] = nxt[b][0][pl.ds(q * 16, 16)]
            idx[b][1][pl.ds(q * 16, 16)] = nxt[b][1][pl.ds(q * 16, 16)]
        fire(b, i)
        prefetch_idx(b, i + 2)

    def wait_gathers(b):
        pltpu.make_async_copy(u_hbm.at[idx[b][0]], us[b], sem_g[b]).wait()
        pltpu.make_async_copy(u_hbm.at[idx[b][1]], ut[b], sem_g[b]).wait()

    def wait_w(b):
        pltpu.make_async_copy(w_hbm.at[pl.ds(0, C)], wv[b], sem_w[b]).wait()

    def compute(b, flag=None):
        wait_gathers(b)
        wait_w(b)

        def group_body(gi, gcarry):
            e0 = gi * 16

            for jj in range(16):
                e = e0 + jj
                acc_a = zf
                acc_b = zf
                acc_c = zf
                for k in range(8):
                    xs = us[b][e, pl.ds(k * 16, 16)]
                    xt = ut[b][e, pl.ds(k * 16, 16)]
                    vs = us[b][e, pl.ds(128 + k * 16, 16)]
                    vt = ut[b][e, pl.ds(128 + k * 16, 16)]
                    dd = xt - xs
                    acc_a = acc_a + dd * dd
                    acc_b = acc_b + vs * dd
                    acc_c = acc_c + vt * dd
                pa[jj, pl.ds(0, 16)] = acc_a
                pb[jj, pl.ds(0, 16)] = acc_b
                pc[jj, pl.ds(0, 16)] = acc_c

            ta = zf
            tb = zf
            tc = zf
            for j in range(16):
                jsp = jnp.full((16,), j, jnp.int32)
                ta = ta + plsc.load_gather(pa, [iota16, jsp])
                tb = tb + plsc.load_gather(pb, [iota16, jsp])
                tc = tc + plsc.load_gather(pc, [iota16, jsp])
            inv = 1.0 / jnp.maximum(ta, 1e-6)
            if flag is not None:
                inv = inv * flag
            csv = tb * inv
            ctv = tc * inv

            for jj in range(16):
                e = e0 + jj
                css = csv[jj]
                cts = ctv[jj]
                for g in range(4):
                    wrow = wv[b][e, pl.ds(g * 16, 16)]
                    ws[e, pl.ds(g * 16, 16)] = wrow * css
                    wt[e, pl.ds(g * 16, 16)] = wrow * cts
            return gcarry

        lax.fori_loop(0, C // 16, group_body, 0)
        pltpu.sync_copy(ws, b_sh.at[idx[b][0]], add=True)
        pltpu.sync_copy(wt, b_sh.at[idx[b][1]], add=True)

    pltpu.sync_copy(src_hbm.at[pl.ds(chunk_base(0), C)], idx[0][0])
    pltpu.sync_copy(dst_hbm.at[pl.ds(chunk_base(0), C)], idx[0][1])
    fire(0, 0)
    prefetch_idx(0, 2)
    zero16 = jnp.zeros((16,), jnp.float32)

    def zrow(r, carry):
        for g in range(4):
            rowbuf[r, pl.ds(g * 16, 16)] = zero16
        return carry

    lax.fori_loop(0, RB, zrow, 0)
    row0 = sid * rpt

    def zbatch(j, carry):
        pltpu.sync_copy(rowbuf, b_sh.at[pl.ds(row0 + j * RB, RB)])
        return carry

    lax.fori_loop(0, nbatch, zbatch, 0)
    plsc.subcore_barrier()

    pltpu.sync_copy(src_hbm.at[pl.ds(chunk_base(1), C)], idx[1][0])
    pltpu.sync_copy(dst_hbm.at[pl.ds(chunk_base(1), C)], idx[1][1])
    fire(1, 1)
    prefetch_idx(1, 3)

    def pair(j, carry):
        compute(0)
        advance(0, 2 * j + 2)
        compute(1)
        advance(1, 2 * j + 3)
        return carry

    lax.fori_loop(0, npairs, pair, 0)
    compute(0, flag=jnp.where(wid < extra, 1.0, 0.0).astype(jnp.float32))
    wait_gathers(1)
    wait_w(1)
    wait_idx(0)
    wait_idx(1)
    plsc.subcore_barrier()

    perms = []
    for g in range(4):
        fl = iota16 + (g * 16)
        i8 = fl // 8
        j8 = fl % 8
        perms.append(j8 * 8 + i8)

    def abatch(j, carry):
        pltpu.sync_copy(b_sh.at[pl.ds(row0 + j * RB, RB)], rowbuf)

        def arow(r, rcarry):
            rsplat = jnp.full((16,), r, jnp.int32)
            for g in range(4):
                bvec = rowbuf[r, pl.ds(g * 16, 16)]
                btv = plsc.load_gather(rowbuf, [rsplat, perms[g]])
                ov[pl.ds(r * 64 + g * 16, 16)] = 0.5 * (bvec - btv)
            return rcarry

        lax.fori_loop(0, RB, arow, 0)
        pltpu.sync_copy(
            ov,
            out_hbm.at[pl.ds((cid * N + row0 + j * RB) * 64, RB * 64)])
        return carry

    lax.fori_loop(0, nbatch, abatch, 0)


@functools.lru_cache(maxsize=None)
def _make_sc(N, E):
    body = functools.partial(_sc_body, N, E)
    return pl.kernel(
        body,
        out_type=jax.ShapeDtypeStruct((NC * N * 64,), jnp.float32),
        mesh=plsc.VectorSubcoreMesh(
            core_axis_name="c", subcore_axis_name="s",
            num_cores=NC, num_subcores=NS),
        scratch_types=[
            pltpu.VMEM((C,), jnp.int32),
            pltpu.VMEM((C,), jnp.int32),
            pltpu.VMEM((C,), jnp.int32),
            pltpu.VMEM((C,), jnp.int32),
            pltpu.VMEM((C,), jnp.int32),
            pltpu.VMEM((C,), jnp.int32),
            pltpu.VMEM((C,), jnp.int32),
            pltpu.VMEM((C,), jnp.int32),
            pltpu.VMEM((C, 256), jnp.float32),
            pltpu.VMEM((C, 256), jnp.float32),
            pltpu.VMEM((C, 256), jnp.float32),
            pltpu.VMEM((C, 256), jnp.float32),
            pltpu.VMEM((C, 64), jnp.float32),
            pltpu.VMEM((C, 64), jnp.float32),
            pltpu.VMEM((C, 64), jnp.float32),
            pltpu.VMEM((C, 64), jnp.float32),
            pltpu.VMEM((16, 17), jnp.float32),
            pltpu.VMEM((16, 17), jnp.float32),
            pltpu.VMEM((16, 17), jnp.float32),
            pltpu.VMEM((RB, 64), jnp.float32),
            pltpu.VMEM((RB * 64,), jnp.float32),
            pltpu.VMEM_SHARED((N, 64), jnp.float32),
            pltpu.SemaphoreType.DMA,
            pltpu.SemaphoreType.DMA,
            pltpu.SemaphoreType.DMA,
            pltpu.SemaphoreType.DMA,
            pltpu.SemaphoreType.DMA,
            pltpu.SemaphoreType.DMA,
        ],
        compiler_params=pltpu.CompilerParams(
            use_tc_tiling_on_sc=False, needs_layout_passes=False),
    )


def _combine_body(p_ref, o_ref):
    o_ref[...] = p_ref[0] + p_ref[1]


@functools.lru_cache(maxsize=None)
def _make_combine(R):
    br = R // 5
    return pl.pallas_call(
        _combine_body,
        out_shape=jax.ShapeDtypeStruct((R, 128), jnp.float32),
        grid=(R // br,),
        in_specs=[pl.BlockSpec((2, br, 128), lambda i: (0, i, 0))],
        out_specs=pl.BlockSpec((br, 128), lambda i: (i, 0)),
    )


@jax.jit
def kernel(x, v, edges, omega_params):
    N, D = x.shape
    E = edges.shape[0]
    K = omega_params.shape[1]
    assert D == 128 and K == 8
    assert E % C == 0 and N % (NS * RB) == 0
    assert ((E // C) // NW) % 2 == 0

    u = jnp.concatenate([x, v], axis=1)
    src = edges[:, 0]
    dst = edges[:, 1]
    w = omega_params.reshape(E, K * K)

    partial = _make_sc(N, E)(u, src, dst, w)
    r = (N * K * K) // 128
    summed = _make_combine(r)(partial.reshape(2, r, 128))
    return summed.reshape(N, K, K)


# speedup vs baseline: 1.1810x; 1.0070x over previous
import functools

import jax
import jax.numpy as jnp
from jax import lax
from jax.experimental import pallas as pl
from jax.experimental.pallas import tpu as pltpu
from jax.experimental.pallas import tpu_sc as plsc

NC = 2
NS = 16
NW = NC * NS
C = 64
RB = 25


def _sc_body(N, E, u_hbm, src_hbm, dst_hbm, w_hbm, out_hbm,
             idx_s0, idx_t0, idx_s1, idx_t1,
             nxt_s0, nxt_t0, nxt_s1, nxt_t1, us0, ut0, us1, ut1,
             wv0, wv1, ws, wt, pa, pb, pc,
             rowbuf, ov, b_sh, sem_g0, sem_g1, sem_w0, sem_w1,
             sem_i0, sem_i1):
    nchunk_tot = E // C
    main = nchunk_tot // NW
    extra = nchunk_tot % NW
    npairs = main // 2
    rpt = N // NS
    nbatch = rpt // RB
    cid = lax.axis_index("c")
    sid = lax.axis_index("s")
    wid = sid * NC + cid
    iota16 = lax.iota(jnp.int32, 16)
    zf = jnp.zeros((16,), jnp.float32)

    idx = ((idx_s0, idx_t0), (idx_s1, idx_t1))
    nxt = ((nxt_s0, nxt_t0), (nxt_s1, nxt_t1))
    us = (us0, us1)
    ut = (ut0, ut1)
    wv = (wv0, wv1)
    sem_g = (sem_g0, sem_g1)
    sem_w = (sem_w0, sem_w1)
    sem_i = (sem_i0, sem_i1)

    def chunk_base(i):
        return jnp.where(i < main, (wid * main + i) * C,
                         jnp.minimum((main * NW + wid) * C, E - C))

    def fire(b, i):
        base = chunk_base(i)
        pltpu.async_copy(u_hbm.at[idx[b][0]], us[b], sem_g[b])
        pltpu.async_copy(u_hbm.at[idx[b][1]], ut[b], sem_g[b])
        pltpu.async_copy(w_hbm.at[pl.ds(base, C)], wv[b], sem_w[b])

    def prefetch_idx(b, i):
        base = chunk_base(i)
        pltpu.async_copy(src_hbm.at[pl.ds(base, C)], nxt[b][0], sem_i[b])
        pltpu.async_copy(dst_hbm.at[pl.ds(base, C)], nxt[b][1], sem_i[b])

    def wait_idx(b):
        pltpu.make_async_copy(
            src_hbm.at[pl.ds(0, C)], nxt[b][0], sem_i[b]).wait()
        pltpu.make_async_copy(
            dst_hbm.at[pl.ds(0, C)], nxt[b][1], sem_i[b]).wait()

    def advance(b, i):
        wait_idx(b)
        for q in range(C // 16):
            idx[b][0][pl.ds(q * 16, 16)] = nxt[b][0][pl.ds(q * 16, 16)]
            idx[b][1][pl.ds(q * 16, 16)] = nxt[b][1][pl.ds(q * 16, 16)]
        fire(b, i)
        prefetch_idx(b, i + 2)

    def wait_gathers(b):
        pltpu.make_async_copy(u_hbm.at[idx[b][0]], us[b], sem_g[b]).wait()
        pltpu.make_async_copy(u_hbm.at[idx[b][1]], ut[b], sem_g[b]).wait()

    def wait_w(b):
        pltpu.make_async_copy(w_hbm.at[pl.ds(0, C)], wv[b], sem_w[b]).wait()

    def compute(b, flag=None):
        wait_gathers(b)
        wait_w(b)

        def group_body(gi, gcarry):
            e0 = gi * 16

            for jj in range(16):
                e = e0 + jj
                a0 = zf
                a1 = zf
                b0 = zf
                b1 = zf
                c0 = zf
                c1 = zf
                for k in range(4):
                    xs2 = us[b][e, pl.ds(k * 32, 32)]
                    xt2 = ut[b][e, pl.ds(k * 32, 32)]
                    vs2 = us[b][e, pl.ds(128 + k * 32, 32)]
                    vt2 = ut[b][e, pl.ds(128 + k * 32, 32)]
                    fmt = plsc.PackFormat.INTERLEAVED
                    xs0, xs1 = plsc.unpack(
                        xs2, format=fmt, preferred_element_type=jnp.float32)
                    xt0, xt1 = plsc.unpack(
                        xt2, format=fmt, preferred_element_type=jnp.float32)
                    vs0, vs1 = plsc.unpack(
                        vs2, format=fmt, preferred_element_type=jnp.float32)
                    vt0, vt1 = plsc.unpack(
                        vt2, format=fmt, preferred_element_type=jnp.float32)
                    d0 = xt0 - xs0
                    d1 = xt1 - xs1
                    a0 = a0 + d0 * d0
                    a1 = a1 + d1 * d1
                    b0 = b0 + vs0 * d0
                    b1 = b1 + vs1 * d1
                    c0 = c0 + vt0 * d0
                    c1 = c1 + vt1 * d1
                pa[jj, pl.ds(0, 16)] = a0 + a1
                pb[jj, pl.ds(0, 16)] = b0 + b1
                pc[jj, pl.ds(0, 16)] = c0 + c1

            ta = zf
            tb = zf
            tc = zf
            for j in range(16):
                jsp = jnp.full((16,), j, jnp.int32)
                ta = ta + plsc.load_gather(pa, [iota16, jsp])
                tb = tb + plsc.load_gather(pb, [iota16, jsp])
                tc = tc + plsc.load_gather(pc, [iota16, jsp])
            inv = 1.0 / jnp.maximum(ta, 1e-6)
            if flag is not None:
                inv = inv * flag
            csv = tb * inv
            ctv = tc * inv

            for jj in range(16):
                e = e0 + jj
                css = csv[jj]
                cts = ctv[jj]
                for g in range(4):
                    wrow = wv[b][e, pl.ds(g * 16, 16)]
                    ws[e, pl.ds(g * 16, 16)] = wrow * css
                    wt[e, pl.ds(g * 16, 16)] = wrow * cts
            return gcarry

        lax.fori_loop(0, C // 16, group_body, 0)
        pltpu.sync_copy(ws, b_sh.at[idx[b][0]], add=True)
        pltpu.sync_copy(wt, b_sh.at[idx[b][1]], add=True)

    pltpu.sync_copy(src_hbm.at[pl.ds(chunk_base(0), C)], idx[0][0])
    pltpu.sync_copy(dst_hbm.at[pl.ds(chunk_base(0), C)], idx[0][1])
    fire(0, 0)
    prefetch_idx(0, 2)
    zero16 = jnp.zeros((16,), jnp.float32)

    def zrow(r, carry):
        for g in range(4):
            rowbuf[r, pl.ds(g * 16, 16)] = zero16
        return carry

    lax.fori_loop(0, RB, zrow, 0)
    row0 = sid * rpt

    def zbatch(j, carry):
        pltpu.sync_copy(rowbuf, b_sh.at[pl.ds(row0 + j * RB, RB)])
        return carry

    lax.fori_loop(0, nbatch, zbatch, 0)
    plsc.subcore_barrier()

    pltpu.sync_copy(src_hbm.at[pl.ds(chunk_base(1), C)], idx[1][0])
    pltpu.sync_copy(dst_hbm.at[pl.ds(chunk_base(1), C)], idx[1][1])
    fire(1, 1)
    prefetch_idx(1, 3)

    def pair(j, carry):
        compute(0)
        advance(0, 2 * j + 2)
        compute(1)
        advance(1, 2 * j + 3)
        return carry

    lax.fori_loop(0, npairs, pair, 0)
    compute(0, flag=jnp.where(wid < extra, 1.0, 0.0).astype(jnp.float32))
    wait_gathers(1)
    wait_w(1)
    wait_idx(0)
    wait_idx(1)
    plsc.subcore_barrier()

    perms = []
    for g in range(4):
        fl = iota16 + (g * 16)
        i8 = fl // 8
        j8 = fl % 8
        perms.append(j8 * 8 + i8)

    def abatch(j, carry):
        pltpu.sync_copy(b_sh.at[pl.ds(row0 + j * RB, RB)], rowbuf)

        def arow(r, rcarry):
            rsplat = jnp.full((16,), r, jnp.int32)
            for g in range(4):
                bvec = rowbuf[r, pl.ds(g * 16, 16)]
                btv = plsc.load_gather(rowbuf, [rsplat, perms[g]])
                ov[pl.ds(r * 64 + g * 16, 16)] = 0.5 * (bvec - btv)
            return rcarry

        lax.fori_loop(0, RB, arow, 0)
        pltpu.sync_copy(
            ov,
            out_hbm.at[pl.ds((cid * N + row0 + j * RB) * 64, RB * 64)])
        return carry

    lax.fori_loop(0, nbatch, abatch, 0)


@functools.lru_cache(maxsize=None)
def _make_sc(N, E):
    body = functools.partial(_sc_body, N, E)
    return pl.kernel(
        body,
        out_type=jax.ShapeDtypeStruct((NC * N * 64,), jnp.float32),
        mesh=plsc.VectorSubcoreMesh(
            core_axis_name="c", subcore_axis_name="s",
            num_cores=NC, num_subcores=NS),
        scratch_types=[
            pltpu.VMEM((C,), jnp.int32),
            pltpu.VMEM((C,), jnp.int32),
            pltpu.VMEM((C,), jnp.int32),
            pltpu.VMEM((C,), jnp.int32),
            pltpu.VMEM((C,), jnp.int32),
            pltpu.VMEM((C,), jnp.int32),
            pltpu.VMEM((C,), jnp.int32),
            pltpu.VMEM((C,), jnp.int32),
            pltpu.VMEM((C, 256), jnp.bfloat16),
            pltpu.VMEM((C, 256), jnp.bfloat16),
            pltpu.VMEM((C, 256), jnp.bfloat16),
            pltpu.VMEM((C, 256), jnp.bfloat16),
            pltpu.VMEM((C, 64), jnp.float32),
            pltpu.VMEM((C, 64), jnp.float32),
            pltpu.VMEM((C, 64), jnp.float32),
            pltpu.VMEM((C, 64), jnp.float32),
            pltpu.VMEM((16, 17), jnp.float32),
            pltpu.VMEM((16, 17), jnp.float32),
            pltpu.VMEM((16, 17), jnp.float32),
            pltpu.VMEM((RB, 64), jnp.float32),
            pltpu.VMEM((RB * 64,), jnp.float32),
            pltpu.VMEM_SHARED((N, 64), jnp.float32),
            pltpu.SemaphoreType.DMA,
            pltpu.SemaphoreType.DMA,
            pltpu.SemaphoreType.DMA,
            pltpu.SemaphoreType.DMA,
            pltpu.SemaphoreType.DMA,
            pltpu.SemaphoreType.DMA,
        ],
        compiler_params=pltpu.CompilerParams(
            use_tc_tiling_on_sc=False, needs_layout_passes=False),
    )


def _combine_body(p_ref, o_ref):
    o_ref[...] = p_ref[0] + p_ref[1]


@functools.lru_cache(maxsize=None)
def _make_combine(R):
    br = R // 5
    return pl.pallas_call(
        _combine_body,
        out_shape=jax.ShapeDtypeStruct((R, 128), jnp.float32),
        grid=(R // br,),
        in_specs=[pl.BlockSpec((2, br, 128), lambda i: (0, i, 0))],
        out_specs=pl.BlockSpec((br, 128), lambda i: (i, 0)),
    )


@jax.jit
def kernel(x, v, edges, omega_params):
    N, D = x.shape
    E = edges.shape[0]
    K = omega_params.shape[1]
    assert D == 128 and K == 8
    assert E % C == 0 and N % (NS * RB) == 0
    assert ((E // C) // NW) % 2 == 0

    u = jnp.concatenate([x, v], axis=1).astype(jnp.bfloat16)
    src = edges[:, 0]
    dst = edges[:, 1]
    w = omega_params.reshape(E, K * K)

    partial = _make_sc(N, E)(u, src, dst, w)
    r = (N * K * K) // 128
    summed = _make_combine(r)(partial.reshape(2, r, 128))
    return summed.reshape(N, K, K)


# speedup vs baseline: 1.2318x; 1.0430x over previous
import functools

import jax
import jax.numpy as jnp
from jax import lax
from jax.experimental import pallas as pl
from jax.experimental.pallas import tpu as pltpu
from jax.experimental.pallas import tpu_sc as plsc

NC = 2
NS = 16
NW = NC * NS
C = 64
RB = 25


def _sc_body(N, E, u_hbm, src_hbm, dst_hbm, w_hbm, out_hbm,
             idx_s0, idx_t0, idx_s1, idx_t1,
             nxt_s0, nxt_t0, nxt_s1, nxt_t1,
             sidx_s0, sidx_t0, sidx_s1, sidx_t1, us0, ut0, us1, ut1,
             wv0, wv1, ws0, wt0, ws1, wt1, pa, pb, pc,
             rowbuf, ov, b_sh, sem_g0, sem_g1, sem_w0, sem_w1,
             sem_i0, sem_i1, sem_s0, sem_s1):
    nchunk_tot = E // C
    main = nchunk_tot // NW
    extra = nchunk_tot % NW
    npairs = main // 2
    rpt = N // NS
    nbatch = rpt // RB
    cid = lax.axis_index("c")
    sid = lax.axis_index("s")
    wid = sid * NC + cid
    iota16 = lax.iota(jnp.int32, 16)
    zf = jnp.zeros((16,), jnp.float32)

    idx = ((idx_s0, idx_t0), (idx_s1, idx_t1))
    nxt = ((nxt_s0, nxt_t0), (nxt_s1, nxt_t1))
    sidx = ((sidx_s0, sidx_t0), (sidx_s1, sidx_t1))
    us = (us0, us1)
    ut = (ut0, ut1)
    wv = (wv0, wv1)
    ws = (ws0, ws1)
    wt = (wt0, wt1)
    sem_g = (sem_g0, sem_g1)
    sem_w = (sem_w0, sem_w1)
    sem_i = (sem_i0, sem_i1)
    sem_s = (sem_s0, sem_s1)

    def chunk_base(i):
        return jnp.where(i < main, (wid * main + i) * C,
                         jnp.minimum((main * NW + wid) * C, E - C))

    def fire(b, i):
        base = chunk_base(i)
        pltpu.async_copy(u_hbm.at[idx[b][0]], us[b], sem_g[b])
        pltpu.async_copy(u_hbm.at[idx[b][1]], ut[b], sem_g[b])
        pltpu.async_copy(w_hbm.at[pl.ds(base, C)], wv[b], sem_w[b])

    def prefetch_idx(b, i):
        base = chunk_base(i)
        pltpu.async_copy(src_hbm.at[pl.ds(base, C)], nxt[b][0], sem_i[b])
        pltpu.async_copy(dst_hbm.at[pl.ds(base, C)], nxt[b][1], sem_i[b])

    def wait_idx(b):
        pltpu.make_async_copy(
            src_hbm.at[pl.ds(0, C)], nxt[b][0], sem_i[b]).wait()
        pltpu.make_async_copy(
            dst_hbm.at[pl.ds(0, C)], nxt[b][1], sem_i[b]).wait()

    def advance(b, i):
        wait_idx(b)
        for q in range(C // 16):
            idx[b][0][pl.ds(q * 16, 16)] = nxt[b][0][pl.ds(q * 16, 16)]
            idx[b][1][pl.ds(q * 16, 16)] = nxt[b][1][pl.ds(q * 16, 16)]
        fire(b, i)
        prefetch_idx(b, i + 2)

    def wait_gathers(b):
        pltpu.make_async_copy(u_hbm.at[idx[b][0]], us[b], sem_g[b]).wait()
        pltpu.make_async_copy(u_hbm.at[idx[b][1]], ut[b], sem_g[b]).wait()

    def wait_scatters(b):
        pltpu.make_async_copy(
            ws[b], b_sh.at[sidx[b][0]], sem_s[b]).wait()
        pltpu.make_async_copy(
            wt[b], b_sh.at[sidx[b][1]], sem_s[b]).wait()

    def wait_w(b):
        pltpu.make_async_copy(w_hbm.at[pl.ds(0, C)], wv[b], sem_w[b]).wait()

    def compute(b, flag=None):
        wait_gathers(b)
        wait_w(b)

        def group_body(gi, gcarry):
            e0 = gi * 16

            for jj in range(16):
                e = e0 + jj
                a0 = zf
                a1 = zf
                b0 = zf
                b1 = zf
                c0 = zf
                c1 = zf
                for k in range(4):
                    xs2 = us[b][e, pl.ds(k * 32, 32)]
                    xt2 = ut[b][e, pl.ds(k * 32, 32)]
                    vs2 = us[b][e, pl.ds(128 + k * 32, 32)]
                    vt2 = ut[b][e, pl.ds(128 + k * 32, 32)]
                    fmt = plsc.PackFormat.INTERLEAVED
                    xs0, xs1 = plsc.unpack(
                        xs2, format=fmt, preferred_element_type=jnp.float32)
                    xt0, xt1 = plsc.unpack(
                        xt2, format=fmt, preferred_element_type=jnp.float32)
                    vs0, vs1 = plsc.unpack(
                        vs2, format=fmt, preferred_element_type=jnp.float32)
                    vt0, vt1 = plsc.unpack(
                        vt2, format=fmt, preferred_element_type=jnp.float32)
                    d0 = xt0 - xs0
                    d1 = xt1 - xs1
                    a0 = a0 + d0 * d0
                    a1 = a1 + d1 * d1
                    b0 = b0 + vs0 * d0
                    b1 = b1 + vs1 * d1
                    c0 = c0 + vt0 * d0
                    c1 = c1 + vt1 * d1
                pa[jj, pl.ds(0, 16)] = a0 + a1
                pb[jj, pl.ds(0, 16)] = b0 + b1
                pc[jj, pl.ds(0, 16)] = c0 + c1

            ta = zf
            tb = zf
            tc = zf
            for j in range(16):
                jsp = jnp.full((16,), j, jnp.int32)
                ta = ta + plsc.load_gather(pa, [iota16, jsp])
                tb = tb + plsc.load_gather(pb, [iota16, jsp])
                tc = tc + plsc.load_gather(pc, [iota16, jsp])
            inv = 1.0 / jnp.maximum(ta, 1e-6)
            if flag is not None:
                inv = inv * flag
            csv = tb * inv
            ctv = tc * inv

            for jj in range(16):
                e = e0 + jj
                css = csv[jj]
                cts = ctv[jj]
                for g in range(4):
                    wrow = wv[b][e, pl.ds(g * 16, 16)]
                    ws[b][e, pl.ds(g * 16, 16)] = wrow * css
                    wt[b][e, pl.ds(g * 16, 16)] = wrow * cts
            return gcarry

        wait_scatters(b)
        lax.fori_loop(0, C // 16, group_body, 0)
        for q in range(C // 16):
            sidx[b][0][pl.ds(q * 16, 16)] = idx[b][0][pl.ds(q * 16, 16)]
            sidx[b][1][pl.ds(q * 16, 16)] = idx[b][1][pl.ds(q * 16, 16)]
        pltpu.async_copy(ws[b], b_sh.at[sidx[b][0]], sem_s[b], add=True)
        pltpu.async_copy(wt[b], b_sh.at[sidx[b][1]], sem_s[b], add=True)

    pltpu.sync_copy(src_hbm.at[pl.ds(chunk_base(0), C)], idx[0][0])
    pltpu.sync_copy(dst_hbm.at[pl.ds(chunk_base(0), C)], idx[0][1])
    fire(0, 0)
    prefetch_idx(0, 2)
    pltpu.async_copy(w_hbm.at[pl.ds(0, C)], ws0, sem_s0)
    pltpu.async_copy(w_hbm.at[pl.ds(0, C)], wt0, sem_s0)
    pltpu.async_copy(w_hbm.at[pl.ds(0, C)], ws1, sem_s1)
    pltpu.async_copy(w_hbm.at[pl.ds(0, C)], wt1, sem_s1)
    zero16 = jnp.zeros((16,), jnp.float32)

    def zrow(r, carry):
        for g in range(4):
            rowbuf[r, pl.ds(g * 16, 16)] = zero16
        return carry

    lax.fori_loop(0, RB, zrow, 0)
    row0 = sid * rpt

    def zbatch(j, carry):
        pltpu.sync_copy(rowbuf, b_sh.at[pl.ds(row0 + j * RB, RB)])
        return carry

    lax.fori_loop(0, nbatch, zbatch, 0)
    plsc.subcore_barrier()

    pltpu.sync_copy(src_hbm.at[pl.ds(chunk_base(1), C)], idx[1][0])
    pltpu.sync_copy(dst_hbm.at[pl.ds(chunk_base(1), C)], idx[1][1])
    fire(1, 1)
    prefetch_idx(1, 3)

    def pair(j, carry):
        compute(0)
        advance(0, 2 * j + 2)
        compute(1)
        advance(1, 2 * j + 3)
        return carry

    lax.fori_loop(0, npairs, pair, 0)
    compute(0, flag=jnp.where(wid < extra, 1.0, 0.0).astype(jnp.float32))
    wait_gathers(1)
    wait_w(1)
    wait_idx(0)
    wait_idx(1)
    wait_scatters(0)
    wait_scatters(1)
    plsc.subcore_barrier()

    perms = []
    for g in range(4):
        fl = iota16 + (g * 16)
        i8 = fl // 8
        j8 = fl % 8
        perms.append(j8 * 8 + i8)

    def abatch(j, carry):
        pltpu.sync_copy(b_sh.at[pl.ds(row0 + j * RB, RB)], rowbuf)

        def arow(r, rcarry):
            rsplat = jnp.full((16,), r, jnp.int32)
            for g in range(4):
                bvec = rowbuf[r, pl.ds(g * 16, 16)]
                btv = plsc.load_gather(rowbuf, [rsplat, perms[g]])
                ov[pl.ds(r * 64 + g * 16, 16)] = 0.5 * (bvec - btv)
            return rcarry

        lax.fori_loop(0, RB, arow, 0)
        pltpu.sync_copy(
            ov,
            out_hbm.at[pl.ds((cid * N + row0 + j * RB) * 64, RB * 64)])
        return carry

    lax.fori_loop(0, nbatch, abatch, 0)


@functools.lru_cache(maxsize=None)
def _make_sc(N, E):
    body = functools.partial(_sc_body, N, E)
    return pl.kernel(
        body,
        out_type=jax.ShapeDtypeStruct((NC * N * 64,), jnp.float32),
        mesh=plsc.VectorSubcoreMesh(
            core_axis_name="c", subcore_axis_name="s",
            num_cores=NC, num_subcores=NS),
        scratch_types=[
            pltpu.VMEM((C,), jnp.int32),
            pltpu.VMEM((C,), jnp.int32),
            pltpu.VMEM((C,), jnp.int32),
            pltpu.VMEM((C,), jnp.int32),
            pltpu.VMEM((C,), jnp.int32),
            pltpu.VMEM((C,), jnp.int32),
            pltpu.VMEM((C,), jnp.int32),
            pltpu.VMEM((C,), jnp.int32),
            pltpu.VMEM((C,), jnp.int32),
            pltpu.VMEM((C,), jnp.int32),
            pltpu.VMEM((C,), jnp.int32),
            pltpu.VMEM((C,), jnp.int32),
            pltpu.VMEM((C, 256), jnp.bfloat16),
            pltpu.VMEM((C, 256), jnp.bfloat16),
            pltpu.VMEM((C, 256), jnp.bfloat16),
            pltpu.VMEM((C, 256), jnp.bfloat16),
            pltpu.VMEM((C, 64), jnp.float32),
            pltpu.VMEM((C, 64), jnp.float32),
            pltpu.VMEM((C, 64), jnp.float32),
            pltpu.VMEM((C, 64), jnp.float32),
            pltpu.VMEM((C, 64), jnp.float32),
            pltpu.VMEM((C, 64), jnp.float32),
            pltpu.VMEM((16, 17), jnp.float32),
            pltpu.VMEM((16, 17), jnp.float32),
            pltpu.VMEM((16, 17), jnp.float32),
            pltpu.VMEM((RB, 64), jnp.float32),
            pltpu.VMEM((RB * 64,), jnp.float32),
            pltpu.VMEM_SHARED((N, 64), jnp.float32),
            pltpu.SemaphoreType.DMA,
            pltpu.SemaphoreType.DMA,
            pltpu.SemaphoreType.DMA,
            pltpu.SemaphoreType.DMA,
            pltpu.SemaphoreType.DMA,
            pltpu.SemaphoreType.DMA,
            pltpu.SemaphoreType.DMA,
            pltpu.SemaphoreType.DMA,
        ],
        compiler_params=pltpu.CompilerParams(
            use_tc_tiling_on_sc=False, needs_layout_passes=False),
    )


def _combine_body(p_ref, o_ref):
    o_ref[...] = p_ref[0] + p_ref[1]


@functools.lru_cache(maxsize=None)
def _make_combine(R):
    br = R // 5
    return pl.pallas_call(
        _combine_body,
        out_shape=jax.ShapeDtypeStruct((R, 128), jnp.float32),
        grid=(R // br,),
        in_specs=[pl.BlockSpec((2, br, 128), lambda i: (0, i, 0))],
        out_specs=pl.BlockSpec((br, 128), lambda i: (i, 0)),
    )


@jax.jit
def kernel(x, v, edges, omega_params):
    N, D = x.shape
    E = edges.shape[0]
    K = omega_params.shape[1]
    assert D == 128 and K == 8
    assert E % C == 0 and N % (NS * RB) == 0
    assert ((E // C) // NW) % 2 == 0

    u = jnp.concatenate([x, v], axis=1).astype(jnp.bfloat16)
    src = edges[:, 0]
    dst = edges[:, 1]
    w = omega_params.reshape(E, K * K)

    partial = _make_sc(N, E)(u, src, dst, w)
    r = (N * K * K) // 128
    summed = _make_combine(r)(partial.reshape(2, r, 128))
    return summed.reshape(N, K, K)


# speedup vs baseline: 1.3034x; 1.0581x over previous
import functools

import jax
import jax.numpy as jnp
from jax import lax
from jax.experimental import pallas as pl
from jax.experimental.pallas import tpu as pltpu
from jax.experimental.pallas import tpu_sc as plsc

NC = 2
NS = 16
NW = NC * NS
C = 80
RB = 25


def _sc_body(N, E, u_hbm, src_hbm, dst_hbm, w_hbm, out_hbm,
             idx_s0, idx_t0, idx_s1, idx_t1,
             nxt_s0, nxt_t0, nxt_s1, nxt_t1,
             sidx_s0, sidx_t0, sidx_s1, sidx_t1, us0, ut0, us1, ut1,
             wv0, wv1, ws0, wt0, ws1, wt1, pa, pb, pc,
             rowbuf, rowbuf1, ov, b_sh, sem_g0, sem_g1, sem_w0, sem_w1,
             sem_i0, sem_i1, sem_s0, sem_s1, sem_z, sem_p0, sem_p1):
    nchunk_tot = E // C
    main = (nchunk_tot // NW) // 2 * 2
    extra = nchunk_tot - main * NW
    npairs = main // 2
    rpt = N // NS
    nbatch = rpt // RB
    cid = lax.axis_index("c")
    sid = lax.axis_index("s")
    wid = sid * NC + cid
    iota16 = lax.iota(jnp.int32, 16)
    zf = jnp.zeros((16,), jnp.float32)

    idx = ((idx_s0, idx_t0), (idx_s1, idx_t1))
    nxt = ((nxt_s0, nxt_t0), (nxt_s1, nxt_t1))
    sidx = ((sidx_s0, sidx_t0), (sidx_s1, sidx_t1))
    us = (us0, us1)
    ut = (ut0, ut1)
    wv = (wv0, wv1)
    ws = (ws0, ws1)
    wt = (wt0, wt1)
    sem_g = (sem_g0, sem_g1)
    sem_w = (sem_w0, sem_w1)
    sem_i = (sem_i0, sem_i1)
    sem_s = (sem_s0, sem_s1)

    def chunk_base(i):
        return jnp.where(i < main, (wid * main + i) * C,
                         jnp.minimum((main * NW + wid) * C, E - C))

    def fire(b, i):
        base = chunk_base(i)
        pltpu.async_copy(u_hbm.at[idx[b][0]], us[b], sem_g[b])
        pltpu.async_copy(u_hbm.at[idx[b][1]], ut[b], sem_g[b])
        pltpu.async_copy(w_hbm.at[pl.ds(base, C)], wv[b], sem_w[b])

    def prefetch_idx(b, i):
        base = chunk_base(i)
        pltpu.async_copy(src_hbm.at[pl.ds(base, C)], nxt[b][0], sem_i[b])
        pltpu.async_copy(dst_hbm.at[pl.ds(base, C)], nxt[b][1], sem_i[b])

    def wait_idx(b):
        pltpu.make_async_copy(
            src_hbm.at[pl.ds(0, C)], nxt[b][0], sem_i[b]).wait()
        pltpu.make_async_copy(
            dst_hbm.at[pl.ds(0, C)], nxt[b][1], sem_i[b]).wait()

    def advance(b, i):
        wait_idx(b)
        for q in range(C // 16):
            idx[b][0][pl.ds(q * 16, 16)] = nxt[b][0][pl.ds(q * 16, 16)]
            idx[b][1][pl.ds(q * 16, 16)] = nxt[b][1][pl.ds(q * 16, 16)]
        fire(b, i)
        prefetch_idx(b, i + 2)

    def wait_gathers(b):
        pltpu.make_async_copy(u_hbm.at[idx[b][0]], us[b], sem_g[b]).wait()
        pltpu.make_async_copy(u_hbm.at[idx[b][1]], ut[b], sem_g[b]).wait()

    def wait_scatters(b):
        pltpu.make_async_copy(
            ws[b], b_sh.at[sidx[b][0]], sem_s[b]).wait()
        pltpu.make_async_copy(
            wt[b], b_sh.at[sidx[b][1]], sem_s[b]).wait()

    def wait_w(b):
        pltpu.make_async_copy(w_hbm.at[pl.ds(0, C)], wv[b], sem_w[b]).wait()

    def compute(b, flag=None):
        wait_gathers(b)
        wait_w(b)

        def group_body(gi, gcarry):
            e0 = gi * 16

            for jj in range(16):
                e = e0 + jj
                a0 = zf
                a1 = zf
                b0 = zf
                b1 = zf
                c0 = zf
                c1 = zf
                for k in range(4):
                    xs2 = us[b][e, pl.ds(k * 32, 32)]
                    xt2 = ut[b][e, pl.ds(k * 32, 32)]
                    vs2 = us[b][e, pl.ds(128 + k * 32, 32)]
                    vt2 = ut[b][e, pl.ds(128 + k * 32, 32)]
                    fmt = plsc.PackFormat.INTERLEAVED
                    xs0, xs1 = plsc.unpack(
                        xs2, format=fmt, preferred_element_type=jnp.float32)
                    xt0, xt1 = plsc.unpack(
                        xt2, format=fmt, preferred_element_type=jnp.float32)
                    vs0, vs1 = plsc.unpack(
                        vs2, format=fmt, preferred_element_type=jnp.float32)
                    vt0, vt1 = plsc.unpack(
                        vt2, format=fmt, preferred_element_type=jnp.float32)
                    d0 = xt0 - xs0
                    d1 = xt1 - xs1
                    a0 = a0 + d0 * d0
                    a1 = a1 + d1 * d1
                    b0 = b0 + vs0 * d0
                    b1 = b1 + vs1 * d1
                    c0 = c0 + vt0 * d0
                    c1 = c1 + vt1 * d1
                pa[jj, pl.ds(0, 16)] = a0 + a1
                pb[jj, pl.ds(0, 16)] = b0 + b1
                pc[jj, pl.ds(0, 16)] = c0 + c1

            ta = zf
            tb = zf
            tc = zf
            for j in range(16):
                jsp = jnp.full((16,), j, jnp.int32)
                ta = ta + plsc.load_gather(pa, [iota16, jsp])
                tb = tb + plsc.load_gather(pb, [iota16, jsp])
                tc = tc + plsc.load_gather(pc, [iota16, jsp])
            inv = 1.0 / jnp.maximum(ta, 1e-6)
            if flag is not None:
                inv = inv * flag
            csv = tb * inv
            ctv = tc * inv

            for jj in range(16):
                e = e0 + jj
                css = csv[jj]
                cts = ctv[jj]
                for g in range(4):
                    wrow = wv[b][e, pl.ds(g * 16, 16)]
                    ws[b][e, pl.ds(g * 16, 16)] = wrow * css
                    wt[b][e, pl.ds(g * 16, 16)] = wrow * cts
            return gcarry

        wait_scatters(b)
        lax.fori_loop(0, C // 16, group_body, 0)
        for q in range(C // 16):
            sidx[b][0][pl.ds(q * 16, 16)] = idx[b][0][pl.ds(q * 16, 16)]
            sidx[b][1][pl.ds(q * 16, 16)] = idx[b][1][pl.ds(q * 16, 16)]
        pltpu.async_copy(ws[b], b_sh.at[sidx[b][0]], sem_s[b], add=True)
        pltpu.async_copy(wt[b], b_sh.at[sidx[b][1]], sem_s[b], add=True)

    pltpu.sync_copy(src_hbm.at[pl.ds(chunk_base(0), C)], idx[0][0])
    pltpu.sync_copy(dst_hbm.at[pl.ds(chunk_base(0), C)], idx[0][1])
    fire(0, 0)
    prefetch_idx(0, 2)
    pltpu.async_copy(w_hbm.at[pl.ds(0, C)], ws0, sem_s0)
    pltpu.async_copy(w_hbm.at[pl.ds(0, C)], wt0, sem_s0)
    pltpu.async_copy(w_hbm.at[pl.ds(0, C)], ws1, sem_s1)
    pltpu.async_copy(w_hbm.at[pl.ds(0, C)], wt1, sem_s1)
    zero16 = jnp.zeros((16,), jnp.float32)

    def zrow(r, carry):
        for g in range(4):
            rowbuf[r, pl.ds(g * 16, 16)] = zero16
        return carry

    lax.fori_loop(0, RB, zrow, 0)
    row0 = sid * rpt

    def zbatch(j, carry):
        pltpu.async_copy(rowbuf, b_sh.at[pl.ds(row0 + j * RB, RB)], sem_z)
        return carry

    lax.fori_loop(0, nbatch, zbatch, 0)

    def zdrain(j, carry):
        pltpu.make_async_copy(
            rowbuf, b_sh.at[pl.ds(row0, RB)], sem_z).wait()
        return carry

    lax.fori_loop(0, nbatch, zdrain, 0)
    plsc.subcore_barrier()

    pltpu.sync_copy(src_hbm.at[pl.ds(chunk_base(1), C)], idx[1][0])
    pltpu.sync_copy(dst_hbm.at[pl.ds(chunk_base(1), C)], idx[1][1])
    fire(1, 1)
    prefetch_idx(1, 3)

    def pair(j, carry):
        compute(0)
        advance(0, 2 * j + 2)
        compute(1)
        advance(1, 2 * j + 3)
        return carry

    lax.fori_loop(0, npairs, pair, 0)
    compute(0, flag=jnp.where(wid < extra, 1.0, 0.0).astype(jnp.float32))
    wait_gathers(1)
    wait_w(1)
    wait_idx(0)
    wait_idx(1)
    wait_scatters(0)
    wait_scatters(1)
    plsc.subcore_barrier()

    perms = []
    for g in range(4):
        fl = iota16 + (g * 16)
        i8 = fl // 8
        j8 = fl % 8
        perms.append(j8 * 8 + i8)

    rbb = (rowbuf, rowbuf1)
    sem_p = (sem_p0, sem_p1)
    pltpu.async_copy(b_sh.at[pl.ds(row0, RB)], rbb[0], sem_p[0])
    for j in range(nbatch):
        rb = rbb[j % 2]
        pltpu.make_async_copy(
            b_sh.at[pl.ds(row0, RB)], rb, sem_p[j % 2]).wait()
        if j + 1 < nbatch:
            pltpu.async_copy(
                b_sh.at[pl.ds(row0 + (j + 1) * RB, RB)],
                rbb[(j + 1) % 2], sem_p[(j + 1) % 2])

        def arow(r, rcarry):
            rsplat = jnp.full((16,), r, jnp.int32)
            for g in range(4):
                bvec = rb[r, pl.ds(g * 16, 16)]
                btv = plsc.load_gather(rb, [rsplat, perms[g]])
                ov[pl.ds(r * 64 + g * 16, 16)] = 0.5 * (bvec - btv)
            return rcarry

        lax.fori_loop(0, RB, arow, 0)
        pltpu.sync_copy(
            ov,
            out_hbm.at[pl.ds((cid * N + row0 + j * RB) * 64, RB * 64)])


@functools.lru_cache(maxsize=None)
def _make_sc(N, E):
    body = functools.partial(_sc_body, N, E)
    return pl.kernel(
        body,
        out_type=jax.ShapeDtypeStruct((NC * N * 64,), jnp.float32),
        mesh=plsc.VectorSubcoreMesh(
            core_axis_name="c", subcore_axis_name="s",
            num_cores=NC, num_subcores=NS),
        scratch_types=[
            pltpu.VMEM((C,), jnp.int32),
            pltpu.VMEM((C,), jnp.int32),
            pltpu.VMEM((C,), jnp.int32),
            pltpu.VMEM((C,), jnp.int32),
            pltpu.VMEM((C,), jnp.int32),
            pltpu.VMEM((C,), jnp.int32),
            pltpu.VMEM((C,), jnp.int32),
            pltpu.VMEM((C,), jnp.int32),
            pltpu.VMEM((C,), jnp.int32),
            pltpu.VMEM((C,), jnp.int32),
            pltpu.VMEM((C,), jnp.int32),
            pltpu.VMEM((C,), jnp.int32),
            pltpu.VMEM((C, 256), jnp.bfloat16),
            pltpu.VMEM((C, 256), jnp.bfloat16),
            pltpu.VMEM((C, 256), jnp.bfloat16),
            pltpu.VMEM((C, 256), jnp.bfloat16),
            pltpu.VMEM((C, 64), jnp.float32),
            pltpu.VMEM((C, 64), jnp.float32),
            pltpu.VMEM((C, 64), jnp.float32),
            pltpu.VMEM((C, 64), jnp.float32),
            pltpu.VMEM((C, 64), jnp.float32),
            pltpu.VMEM((C, 64), jnp.float32),
            pltpu.VMEM((16, 17), jnp.float32),
            pltpu.VMEM((16, 17), jnp.float32),
            pltpu.VMEM((16, 17), jnp.float32),
            pltpu.VMEM((RB, 64), jnp.float32),
            pltpu.VMEM((RB, 64), jnp.float32),
            pltpu.VMEM((RB * 64,), jnp.float32),
            pltpu.VMEM_SHARED((N, 64), jnp.float32),
            pltpu.SemaphoreType.DMA,
            pltpu.SemaphoreType.DMA,
            pltpu.SemaphoreType.DMA,
            pltpu.SemaphoreType.DMA,
            pltpu.SemaphoreType.DMA,
            pltpu.SemaphoreType.DMA,
            pltpu.SemaphoreType.DMA,
            pltpu.SemaphoreType.DMA,
            pltpu.SemaphoreType.DMA,
            pltpu.SemaphoreType.DMA,
            pltpu.SemaphoreType.DMA,
        ],
        compiler_params=pltpu.CompilerParams(
            use_tc_tiling_on_sc=False, needs_layout_passes=False),
    )


def _combine_body(p_ref, o_ref):
    o_ref[...] = p_ref[0] + p_ref[1]


@functools.lru_cache(maxsize=None)
def _make_combine(R):
    br = R // 5
    return pl.pallas_call(
        _combine_body,
        out_shape=jax.ShapeDtypeStruct((R, 128), jnp.float32),
        grid=(R // br,),
        in_specs=[pl.BlockSpec((2, br, 128), lambda i: (0, i, 0))],
        out_specs=pl.BlockSpec((br, 128), lambda i: (i, 0)),
    )


@jax.jit
def kernel(x, v, edges, omega_params):
    N, D = x.shape
    E = edges.shape[0]
    K = omega_params.shape[1]
    assert D == 128 and K == 8
    assert E % C == 0 and N % (NS * RB) == 0
    _main = (E // C) // NW // 2 * 2
    assert _main >= 2 and (E // C) - _main * NW <= NW

    u = jnp.concatenate([x, v], axis=1).astype(jnp.bfloat16)
    src = edges[:, 0]
    dst = edges[:, 1]
    w = omega_params.reshape(E, K * K)

    partial = _make_sc(N, E)(u, src, dst, w)
    r = (N * K * K) // 128
    summed = _make_combine(r)(partial.reshape(2, r, 128))
    return summed.reshape(N, K, K)


# speedup vs baseline: 1.3094x; 1.0046x over previous
import functools

import jax
import jax.numpy as jnp
from jax import lax
from jax.experimental import pallas as pl
from jax.experimental.pallas import tpu as pltpu
from jax.experimental.pallas import tpu_sc as plsc

NC = 2
NS = 16
NW = NC * NS
C = 80
RB = 25


def _sc_body(N, E, u_hbm, src_hbm, dst_hbm, w_hbm, out_hbm,
             idx_s0, idx_t0, idx_s1, idx_t1,
             nxt_s0, nxt_t0, nxt_s1, nxt_t1,
             sidx_s0, sidx_t0, sidx_s1, sidx_t1, us0, ut0, us1, ut1,
             wv0, wv1, ws0, wt0, ws1, wt1, pa, pb, pc,
             rowbuf, rowbuf1, ov, b_sh, sem_g0, sem_g1, sem_w0, sem_w1,
             sem_i0, sem_i1, sem_s0, sem_s1, sem_z, sem_p0, sem_p1):
    nchunk_tot = E // C
    main = (nchunk_tot // NW) // 2 * 2
    extra = nchunk_tot - main * NW
    npairs = main // 2
    rpt = N // NS
    nbatch = rpt // RB
    cid = lax.axis_index("c")
    sid = lax.axis_index("s")
    wid = sid * NC + cid
    iota16 = lax.iota(jnp.int32, 16)
    zf = jnp.zeros((16,), jnp.float32)

    idx = ((idx_s0, idx_t0), (idx_s1, idx_t1))
    nxt = ((nxt_s0, nxt_t0), (nxt_s1, nxt_t1))
    sidx = ((sidx_s0, sidx_t0), (sidx_s1, sidx_t1))
    us = (us0, us1)
    ut = (ut0, ut1)
    wv = (wv0, wv1)
    ws = (ws0, ws1)
    wt = (wt0, wt1)
    sem_g = (sem_g0, sem_g1)
    sem_w = (sem_w0, sem_w1)
    sem_i = (sem_i0, sem_i1)
    sem_s = (sem_s0, sem_s1)

    def chunk_base(i):
        return jnp.where(i < main, (wid * main + i) * C,
                         jnp.minimum((main * NW + wid) * C, E - C))

    def fire(b, i):
        base = chunk_base(i)
        pltpu.async_copy(u_hbm.at[idx[b][0]], us[b], sem_g[b])
        pltpu.async_copy(u_hbm.at[idx[b][1]], ut[b], sem_g[b])
        pltpu.async_copy(w_hbm.at[pl.ds(base, C)], wv[b], sem_w[b])

    def prefetch_idx(b, i):
        base = chunk_base(i)
        pltpu.async_copy(src_hbm.at[pl.ds(base, C)], nxt[b][0], sem_i[b])
        pltpu.async_copy(dst_hbm.at[pl.ds(base, C)], nxt[b][1], sem_i[b])

    def wait_idx(b):
        pltpu.make_async_copy(
            src_hbm.at[pl.ds(0, C)], nxt[b][0], sem_i[b]).wait()
        pltpu.make_async_copy(
            dst_hbm.at[pl.ds(0, C)], nxt[b][1], sem_i[b]).wait()

    def advance(b, i):
        wait_idx(b)
        for q in range(C // 16):
            idx[b][0][pl.ds(q * 16, 16)] = nxt[b][0][pl.ds(q * 16, 16)]
            idx[b][1][pl.ds(q * 16, 16)] = nxt[b][1][pl.ds(q * 16, 16)]
        fire(b, i)
        prefetch_idx(b, i + 2)

    def wait_gathers(b):
        pltpu.make_async_copy(u_hbm.at[idx[b][0]], us[b], sem_g[b]).wait()
        pltpu.make_async_copy(u_hbm.at[idx[b][1]], ut[b], sem_g[b]).wait()

    def wait_scatters(b):
        pltpu.make_async_copy(
            ws[b], b_sh.at[sidx[b][0]], sem_s[b]).wait()
        pltpu.make_async_copy(
            wt[b], b_sh.at[sidx[b][1]], sem_s[b]).wait()

    def wait_w(b):
        pltpu.make_async_copy(w_hbm.at[pl.ds(0, C)], wv[b], sem_w[b]).wait()

    def compute(b, flag=None):
        wait_gathers(b)
        wait_w(b)

        def group_body(gi, gcarry):
            e0 = gi * 16

            for jj in range(16):
                e = e0 + jj
                a0 = zf
                a1 = zf
                b0 = zf
                b1 = zf
                c0 = zf
                c1 = zf
                for k in range(4):
                    xs2 = us[b][e, pl.ds(k * 32, 32)]
                    xt2 = ut[b][e, pl.ds(k * 32, 32)]
                    vs2 = us[b][e, pl.ds(128 + k * 32, 32)]
                    vt2 = ut[b][e, pl.ds(128 + k * 32, 32)]
                    fmt = plsc.PackFormat.INTERLEAVED
                    xs0, xs1 = plsc.unpack(
                        xs2, format=fmt, preferred_element_type=jnp.float32)
                    xt0, xt1 = plsc.unpack(
                        xt2, format=fmt, preferred_element_type=jnp.float32)
                    vs0, vs1 = plsc.unpack(
                        vs2, format=fmt, preferred_element_type=jnp.float32)
                    vt0, vt1 = plsc.unpack(
                        vt2, format=fmt, preferred_element_type=jnp.float32)
                    d0 = xt0 - xs0
                    d1 = xt1 - xs1
                    a0 = a0 + d0 * d0
                    a1 = a1 + d1 * d1
                    b0 = b0 + vs0 * d0
                    b1 = b1 + vs1 * d1
                    c0 = c0 + vt0 * d0
                    c1 = c1 + vt1 * d1
                pa[jj, pl.ds(0, 16)] = a0 + a1
                pb[jj, pl.ds(0, 16)] = b0 + b1
                pc[jj, pl.ds(0, 16)] = c0 + c1

            ta = zf
            tb = zf
            tc = zf
            for j in range(16):
                jsp = jnp.full((16,), j, jnp.int32)
                ta = ta + plsc.load_gather(pa, [iota16, jsp])
                tb = tb + plsc.load_gather(pb, [iota16, jsp])
                tc = tc + plsc.load_gather(pc, [iota16, jsp])
            inv = 1.0 / jnp.maximum(ta, 1e-6)
            if flag is not None:
                inv = inv * flag
            csv = tb * inv
            ctv = tc * inv

            for jj in range(16):
                e = e0 + jj
                css = csv[jj]
                cts = ctv[jj]
                for g in range(4):
                    wrow = wv[b][e, pl.ds(g * 16, 16)]
                    ws[b][e, pl.ds(g * 16, 16)] = wrow * css
                    wt[b][e, pl.ds(g * 16, 16)] = wrow * cts
            return gcarry

        wait_scatters(b)
        lax.fori_loop(0, C // 16, group_body, 0)
        for q in range(C // 16):
            sidx[b][0][pl.ds(q * 16, 16)] = idx[b][0][pl.ds(q * 16, 16)]
            sidx[b][1][pl.ds(q * 16, 16)] = idx[b][1][pl.ds(q * 16, 16)]
        pltpu.async_copy(ws[b], b_sh.at[sidx[b][0]], sem_s[b], add=True)
        pltpu.async_copy(wt[b], b_sh.at[sidx[b][1]], sem_s[b], add=True)

    pltpu.sync_copy(src_hbm.at[pl.ds(chunk_base(0), C)], idx[0][0])
    pltpu.sync_copy(dst_hbm.at[pl.ds(chunk_base(0), C)], idx[0][1])
    fire(0, 0)
    prefetch_idx(0, 2)
    pltpu.async_copy(w_hbm.at[pl.ds(0, C)], ws0, sem_s0)
    pltpu.async_copy(w_hbm.at[pl.ds(0, C)], wt0, sem_s0)
    pltpu.async_copy(w_hbm.at[pl.ds(0, C)], ws1, sem_s1)
    pltpu.async_copy(w_hbm.at[pl.ds(0, C)], wt1, sem_s1)
    zero16 = jnp.zeros((16,), jnp.float32)

    def zrow(r, carry):
        for g in range(4):
            rowbuf[r, pl.ds(g * 16, 16)] = zero16
        return carry

    lax.fori_loop(0, RB, zrow, 0)
    row0 = sid * rpt

    def zbatch(j, carry):
        pltpu.async_copy(rowbuf, b_sh.at[pl.ds(row0 + j * RB, RB)], sem_z)
        return carry

    lax.fori_loop(0, nbatch, zbatch, 0)

    def zdrain(j, carry):
        pltpu.make_async_copy(
            rowbuf, b_sh.at[pl.ds(row0, RB)], sem_z).wait()
        return carry

    lax.fori_loop(0, nbatch, zdrain, 0)
    plsc.subcore_barrier()

    pltpu.sync_copy(src_hbm.at[pl.ds(chunk_base(1), C)], idx[1][0])
    pltpu.sync_copy(dst_hbm.at[pl.ds(chunk_base(1), C)], idx[1][1])
    fire(1, 1)
    prefetch_idx(1, 3)

    def pair(j, carry):
        compute(0)
        advance(0, 2 * j + 2)
        compute(1)
        advance(1, 2 * j + 3)
        return carry

    lax.fori_loop(0, npairs, pair, 0)
    compute(0, flag=jnp.where(wid < extra, 1.0, 0.0).astype(jnp.float32))
    wait_gathers(1)
    wait_w(1)
    wait_idx(0)
    wait_idx(1)
    wait_scatters(0)
    wait_scatters(1)
    plsc.subcore_barrier()

    perms = []
    for g in range(4):
        fl = iota16 + (g * 16)
        i8 = fl // 8
        j8 = fl % 8
        perms.append(j8 * 8 + i8)

    rbb = (rowbuf, rowbuf1)
    sem_p = (sem_p0, sem_p1)
    pltpu.async_copy(b_sh.at[pl.ds(row0, RB)], rbb[0], sem_p[0])
    for j in range(nbatch):
        rb = rbb[j % 2]
        pltpu.make_async_copy(
            b_sh.at[pl.ds(row0, RB)], rb, sem_p[j % 2]).wait()
        if j + 1 < nbatch:
            pltpu.async_copy(
                b_sh.at[pl.ds(row0 + (j + 1) * RB, RB)],
                rbb[(j + 1) % 2], sem_p[(j + 1) % 2])

        def arow(r, rcarry):
            rsplat = jnp.full((16,), r, jnp.int32)
            for g in range(4):
                bvec = rb[r, pl.ds(g * 16, 16)]
                btv = plsc.load_gather(rb, [rsplat, perms[g]])
                ov[pl.ds(r * 64 + g * 16, 16)] = 0.5 * (bvec - btv)
            return rcarry

        lax.fori_loop(0, RB, arow, 0)
        pltpu.sync_copy(
            ov,
            out_hbm.at[pl.ds((cid * N + row0 + j * RB) * 64, RB * 64)])


@functools.lru_cache(maxsize=None)
def _make_sc(N, E):
    body = functools.partial(_sc_body, N, E)
    return pl.kernel(
        body,
        out_type=jax.ShapeDtypeStruct((NC * N * 64,), jnp.float32),
        mesh=plsc.VectorSubcoreMesh(
            core_axis_name="c", subcore_axis_name="s",
            num_cores=NC, num_subcores=NS),
        scratch_types=[
            pltpu.VMEM((C,), jnp.int32),
            pltpu.VMEM((C,), jnp.int32),
            pltpu.VMEM((C,), jnp.int32),
            pltpu.VMEM((C,), jnp.int32),
            pltpu.VMEM((C,), jnp.int32),
            pltpu.VMEM((C,), jnp.int32),
            pltpu.VMEM((C,), jnp.int32),
            pltpu.VMEM((C,), jnp.int32),
            pltpu.VMEM((C,), jnp.int32),
            pltpu.VMEM((C,), jnp.int32),
            pltpu.VMEM((C,), jnp.int32),
            pltpu.VMEM((C,), jnp.int32),
            pltpu.VMEM((C, 256), jnp.bfloat16),
            pltpu.VMEM((C, 256), jnp.bfloat16),
            pltpu.VMEM((C, 256), jnp.bfloat16),
            pltpu.VMEM((C, 256), jnp.bfloat16),
            pltpu.VMEM((C, 64), jnp.float32),
            pltpu.VMEM((C, 64), jnp.float32),
            pltpu.VMEM((C, 64), jnp.float32),
            pltpu.VMEM((C, 64), jnp.float32),
            pltpu.VMEM((C, 64), jnp.float32),
            pltpu.VMEM((C, 64), jnp.float32),
            pltpu.VMEM((16, 17), jnp.float32),
            pltpu.VMEM((16, 17), jnp.float32),
            pltpu.VMEM((16, 17), jnp.float32),
            pltpu.VMEM((RB, 64), jnp.float32),
            pltpu.VMEM((RB, 64), jnp.float32),
            pltpu.VMEM((RB * 64,), jnp.float32),
            pltpu.VMEM_SHARED((N, 64), jnp.float32),
            pltpu.SemaphoreType.DMA,
            pltpu.SemaphoreType.DMA,
            pltpu.SemaphoreType.DMA,
            pltpu.SemaphoreType.DMA,
            pltpu.SemaphoreType.DMA,
            pltpu.SemaphoreType.DMA,
            pltpu.SemaphoreType.DMA,
            pltpu.SemaphoreType.DMA,
            pltpu.SemaphoreType.DMA,
            pltpu.SemaphoreType.DMA,
            pltpu.SemaphoreType.DMA,
        ],
        compiler_params=pltpu.CompilerParams(
            use_tc_tiling_on_sc=False, needs_layout_passes=False),
    )


def _concat_body(x_ref, v_ref, o_ref):
    o_ref[:, 0:128] = x_ref[...].astype(jnp.bfloat16)
    o_ref[:, 128:256] = v_ref[...].astype(jnp.bfloat16)


@functools.lru_cache(maxsize=None)
def _make_concat(n):
    bn = 400
    return pl.pallas_call(
        _concat_body,
        out_shape=jax.ShapeDtypeStruct((n, 256), jnp.bfloat16),
        grid=(n // bn,),
        in_specs=[pl.BlockSpec((bn, 128), lambda i: (i, 0)),
                  pl.BlockSpec((bn, 128), lambda i: (i, 0))],
        out_specs=pl.BlockSpec((bn, 256), lambda i: (i, 0)),
    )


def _combine_body(p_ref, o_ref):
    o_ref[...] = p_ref[0] + p_ref[1]


@functools.lru_cache(maxsize=None)
def _make_combine(R):
    br = R // 5
    return pl.pallas_call(
        _combine_body,
        out_shape=jax.ShapeDtypeStruct((R, 128), jnp.float32),
        grid=(R // br,),
        in_specs=[pl.BlockSpec((2, br, 128), lambda i: (0, i, 0))],
        out_specs=pl.BlockSpec((br, 128), lambda i: (i, 0)),
    )


@jax.jit
def kernel(x, v, edges, omega_params):
    N, D = x.shape
    E = edges.shape[0]
    K = omega_params.shape[1]
    assert D == 128 and K == 8
    assert E % C == 0 and N % (NS * RB) == 0
    _main = (E // C) // NW // 2 * 2
    assert _main >= 2 and (E // C) - _main * NW <= NW

    u = _make_concat(N)(x, v)
    src = edges[:, 0]
    dst = edges[:, 1]
    w = omega_params.reshape(E, K * K)

    partial = _make_sc(N, E)(u, src, dst, w)
    r = (N * K * K) // 128
    summed = _make_combine(r)(partial.reshape(2, r, 128))
    return summed.reshape(N, K, K)
